# baseline (speedup 1.0000x reference)
import functools

import jax
import jax.numpy as jnp
from jax import lax
from jax.experimental import pallas as pl
from jax.experimental.pallas import tpu as pltpu
from jax.experimental.pallas import tpu_sc as plsc

NC = 2
NS = 16
NW = NC * NS
L = 16

BATCH = 16384
HIST = 20
DIM = 64
B = BATCH * HIST
VPAIR = 500000
BW = BATCH // NW
B_PER_W = BW * HIST
CHUNK = 256
NBUF = 2
NSTEPS = B_PER_W // CHUNK
NROUNDS = NSTEPS // NBUF


@functools.partial(
    pl.kernel,
    out_type=jax.ShapeDtypeStruct((HIST, DIM, BATCH), jnp.float32),
    mesh=plsc.VectorSubcoreMesh(core_axis_name="c", subcore_axis_name="s"),
    scratch_types=[
        pltpu.VMEM((HIST, BW), jnp.int32),
        pltpu.VMEM((B_PER_W,), jnp.int32),
        pltpu.VMEM((B_PER_W,), jnp.int32),
        pltpu.VMEM((NBUF, CHUNK, 2 * DIM), jnp.float32),
        pltpu.VMEM((2, DIM, 2 * DIM), jnp.float32),
        pltpu.SemaphoreType.DMA((NBUF,)),
        pltpu.SemaphoreType.DMA((2,)),
    ],
    compiler_params=pltpu.CompilerParams(
        use_tc_tiling_on_sc=True, needs_layout_passes=False),
)
def _embed_kernel(wpair_hbm, it_hbm, out_hbm, idx_v, gidx_v, off_v, pairs_v,
                  stage_v, gsem, ssem):
    wid = lax.axis_index("s") * NC + lax.axis_index("c")
    b0 = wid * BW

    pltpu.sync_copy(it_hbm.at[:, pl.ds(b0, BW)], idx_v)

    @pl.loop(0, HIST)
    def _h(h):
        @pl.loop(0, BW // L)
        def _g(g):
            v = idx_v[h, pl.ds(g * L, L)]
            gidx_v[pl.ds(h * BW + g * L, L)] = v >> 1
            off_v[pl.ds(h * BW + g * L, L)] = (v & 1) << 6

    def start_gather(k, b):
        pltpu.async_copy(
            wpair_hbm.at[gidx_v.at[pl.ds(k * CHUNK, CHUNK)]],
            pairs_v.at[b],
            gsem.at[b],
        )

    iota = lax.iota(jnp.int32, L)

    def extract(k, b, sb):
        @pl.loop(0, 8)
        def _bb(bb):
            svec = 128 * sb + 16 * bb + iota
            ovec = off_v[pl.ds(k * CHUNK + 128 * sb + 16 * bb, L)]
            for d in range(DIM):
                val = plsc.load_gather(pairs_v.at[b], [svec, ovec + d])
                stage_v[sb, d, pl.ds(16 * bb, L)] = val

    for b in range(NBUF):
        start_gather(b, b)

    @pl.loop(0, NROUNDS)
    def _round(g):
        for b in range(NBUF):
            k = g * NBUF + b
            pltpu.make_async_copy(
                wpair_hbm.at[gidx_v.at[pl.ds(0, CHUNK)]], pairs_v.at[b],
                gsem.at[b],
            ).wait()
            for sb in range(2):
                @pl.when(k > 0)
                def _():
                    pltpu.make_async_copy(
                        stage_v.at[sb], out_hbm.at[0, :, pl.ds(0, 2 * DIM)],
                        ssem.at[sb],
                    ).wait()
                extract(k, b, sb)
                pltpu.async_copy(
                    stage_v.at[sb],
                    out_hbm.at[g, :, pl.ds(b0 + 256 * b + 128 * sb, 2 * DIM)],
                    ssem.at[sb],
                )
            @pl.when(k + NBUF < NSTEPS)
            def _():
                start_gather(k + NBUF, b)

    for sb in range(2):
        pltpu.make_async_copy(
            stage_v.at[sb], out_hbm.at[0, :, pl.ds(0, 2 * DIM)], ssem.at[sb]
        ).wait()


def kernel(input_, weight):
    it = input_.T.astype(jnp.int32)
    wpair = weight.reshape(VPAIR, 2 * DIM)
    out_t = _embed_kernel(wpair, it)
    return jnp.transpose(out_t, (2, 0, 1))
